# Initial kernel scaffold; baseline (speedup 1.0000x reference)
import jax
import jax.numpy as jnp
from jax.experimental import pallas as pl


def kernel(x_nchw, wconv_pt, bconv, whead_pt, bhead):
    raise NotImplementedError("write your pallas kernel here")


# speedup vs baseline: 7.4568x; 7.4568x over previous
import jax
import jax.numpy as jnp
from jax.experimental import pallas as pl
from jax.experimental.pallas import tpu as pltpu


def _round_up(x, m):
    return (x + m - 1) // m * m


def kernel(x_nchw, wconv_pt, bconv, whead_pt, bhead):
    B, C, H, W = x_nchw.shape
    F = wconv_pt.shape[0]
    n_class = whead_pt.shape[0]
    R = H * W
    KC = 3 * C
    KL = _round_up(KC, 16)
    K = 3 * KL
    F_pad = _round_up(F, 128)
    C_pad = _round_up(n_class, 128)

    x_nhwc = jnp.transpose(x_nchw, (0, 2, 3, 1))
    xpw = jnp.pad(x_nhwc, ((0, 0), (0, 0), (1, 1), (0, 0)))
    taps = jnp.stack([xpw[:, :, kx:kx + W, :] for kx in range(3)], axis=3)
    xrow = taps.reshape(B, R, KC)
    xrow = jnp.pad(xrow, ((0, 0), (W, W), (0, KL - KC)))
    xrow = xrow.astype(jnp.bfloat16)

    wk = jnp.transpose(wconv_pt, (2, 3, 1, 0)).reshape(3, KC, F)
    wk = jnp.pad(wk, ((0, 0), (0, KL - KC), (0, F_pad - F)))
    wk = wk.reshape(K, F_pad).astype(jnp.bfloat16)

    bconv_p = jnp.pad(bconv.reshape(1, F),
                      ((0, 0), (0, F_pad - F))).astype(jnp.float32)
    whead = (jnp.transpose(whead_pt, (1, 0)) / float(R))
    whead = jnp.pad(whead, ((0, F_pad - F),
                            (0, C_pad - n_class))).astype(jnp.float32)
    bhead_p = jnp.pad(bhead.reshape(1, n_class),
                      ((0, 0), (0, C_pad - n_class))).astype(jnp.float32)

    def _body(xr_ref, w_ref, bc_ref, wh_ref, bh_ref, out_ref):
        blk = xr_ref[0]
        patches = jnp.concatenate(
            [blk[0:R], blk[W:W + R], blk[2 * W:2 * W + R]], axis=1)
        conv = jnp.dot(patches, w_ref[...],
                       preferred_element_type=jnp.float32)
        conv = jnp.maximum(conv + bc_ref[...], 0.0)
        pooled = jnp.sum(conv, axis=0, keepdims=True)
        logits = jnp.dot(pooled, wh_ref[...],
                         preferred_element_type=jnp.float32) + bh_ref[...]
        out_ref[0] = logits

    flops = 2 * B * R * K * F_pad + 2 * B * F_pad * C_pad
    bytes_accessed = (xrow.size * 2 + wk.size * 2
                      + (bconv_p.size + whead.size + bhead_p.size) * 4
                      + B * C_pad * 4)

    out = pl.pallas_call(
        _body,
        out_shape=jax.ShapeDtypeStruct((B, 1, C_pad), jnp.float32),
        grid=(B,),
        in_specs=[
            pl.BlockSpec((1, R + 2 * W, KL), lambda b: (b, 0, 0)),
            pl.BlockSpec((K, F_pad), lambda b: (0, 0)),
            pl.BlockSpec((1, F_pad), lambda b: (0, 0)),
            pl.BlockSpec((F_pad, C_pad), lambda b: (0, 0)),
            pl.BlockSpec((1, C_pad), lambda b: (0, 0)),
        ],
        out_specs=pl.BlockSpec((1, 1, C_pad), lambda b: (b, 0, 0)),
        compiler_params=pltpu.CompilerParams(
            dimension_semantics=("parallel",),
            vmem_limit_bytes=48 * 1024 * 1024,
        ),
        cost_estimate=pl.CostEstimate(
            flops=flops, transcendentals=0, bytes_accessed=bytes_accessed),
    )(xrow, wk, bconv_p, whead, bhead_p)

    return out[:, 0, :n_class]
